# baseline (speedup 1.0000x reference)
import math

import jax
import jax.numpy as jnp
import numpy as np
from jax.experimental import pallas as pl
from jax.experimental.pallas import tpu as pltpu

_D_IN = 512
_H = 1024
_TOPK = 1024
_DECAY = 0.99
_NEG_INF = np.float32(-np.inf)
_INT_MIN = np.int32(-2147483648)


def _alpha_body(fs_ref, f_ref, o_ref):
    o_ref[...] = jax.lax.dot_general(
        fs_ref[...], f_ref[...], (((1,), (0,)), ((), ())),
        preferred_element_type=jnp.float32)


def _compute_alpha(fs2, feature):
    n = fs2.shape[0]
    blk = 2000
    return pl.pallas_call(
        _alpha_body,
        grid=(n // blk,),
        in_specs=[
            pl.BlockSpec((blk, _D_IN), lambda i: (i, 0)),
            pl.BlockSpec((_D_IN, 1), lambda i: (0, 0)),
        ],
        out_specs=pl.BlockSpec((blk, 1), lambda i: (i, 0)),
        out_shape=jax.ShapeDtypeStruct((n, 1), jnp.float32),
    )(fs2, feature.reshape(_D_IN, 1))


def _weights_body(a_ref, ts_ref, t_ref, w_ref):
    a = a_ref[...]
    rows, lanes = a.shape
    k = _TOPK

    u = jax.lax.bitcast_convert_type(a, jnp.int32)
    key = jnp.where(u < 0, u ^ np.int32(0x7FFFFFFF), u)

    p_u = np.int32(0)
    for b in range(31, -1, -1):
        c_u = p_u | np.int32((1 << b) - (1 << 32 if b == 31 else 0))
        c_s = c_u ^ _INT_MIN
        cnt = jnp.sum((key >= c_s).astype(jnp.int32))
        p_u = jnp.where(cnt >= k, c_u, p_u)
    thr = p_u ^ _INT_MIN

    gt = key > thr
    eq = key == thr
    n_gt = jnp.sum(gt.astype(jnp.int32))
    r = k - n_gt

    idx = (jax.lax.broadcasted_iota(jnp.int32, (rows, lanes), 0) * lanes
           + jax.lax.broadcasted_iota(jnp.int32, (rows, lanes), 1))

    tie_idx = np.int32(0)
    for b in range(14, -1, -1):
        c = tie_idx | np.int32(1 << b)
        cnt = jnp.sum((eq & (idx < c)).astype(jnp.int32))
        tie_idx = jnp.where(cnt < r, c, tie_idx)
    sel = gt | (eq & (idx <= tie_idx))

    t = t_ref[0, 0]
    decay = jnp.exp((t - ts_ref[...]) * np.float32(math.log(_DECAY)))
    s = jnp.where(sel, a * decay, _NEG_INF)
    m = jnp.max(s)
    e = jnp.where(sel, jnp.exp(s - m), np.float32(0.0))
    w_ref[...] = e / jnp.sum(e)


def _compute_weights(alpha_pad, ts_pad, t_arr):
    rows = alpha_pad.shape[0]
    return pl.pallas_call(
        _weights_body,
        in_specs=[
            pl.BlockSpec((rows, 128), lambda: (0, 0)),
            pl.BlockSpec((rows, 128), lambda: (0, 0)),
            pl.BlockSpec(memory_space=pltpu.SMEM),
        ],
        out_specs=pl.BlockSpec((rows, 128), lambda: (0, 0)),
        out_shape=jax.ShapeDtypeStruct((rows, 128), jnp.float32),
    )(alpha_pad, ts_pad, t_arr)


def _attn_body(w_ref, hs_ref, o_ref):
    @pl.when(pl.program_id(0) == 0)
    def _():
        o_ref[...] = jnp.zeros_like(o_ref)

    o_ref[...] += jax.lax.dot_general(
        w_ref[...], hs_ref[...], (((0,), (0,)), ((), ())),
        preferred_element_type=jnp.float32)


def _compute_attn(w, hs_flat):
    n = hs_flat.shape[0]
    blk = 2000
    return pl.pallas_call(
        _attn_body,
        grid=(n // blk,),
        in_specs=[
            pl.BlockSpec((blk, 1), lambda i: (i, 0)),
            pl.BlockSpec((blk, _H), lambda i: (i, 0)),
        ],
        out_specs=pl.BlockSpec((1, _H), lambda i: (0, 0)),
        out_shape=jax.ShapeDtypeStruct((1, _H), jnp.float32),
    )(w, hs_flat)


def _gates_body(wi_ref, wh_ref, x_ref, h_ref, bi_ref, bh_ref, gi_ref, gh_ref):
    gi_ref[...] = jax.lax.dot_general(
        wi_ref[...], x_ref[...], (((1,), (0,)), ((), ())),
        preferred_element_type=jnp.float32) + bi_ref[...]
    gh_ref[...] = jax.lax.dot_general(
        wh_ref[...], h_ref[...], (((1,), (0,)), ((), ())),
        preferred_element_type=jnp.float32) + bh_ref[...]


def _compute_gates(W_ih, W_hh, x, h0, b_ih, b_hh):
    rows = W_ih.shape[0]
    dx = W_ih.shape[1]
    blk = 512
    return pl.pallas_call(
        _gates_body,
        grid=(rows // blk,),
        in_specs=[
            pl.BlockSpec((blk, dx), lambda i: (i, 0)),
            pl.BlockSpec((blk, _H), lambda i: (i, 0)),
            pl.BlockSpec((dx, 1), lambda i: (0, 0)),
            pl.BlockSpec((_H, 1), lambda i: (0, 0)),
            pl.BlockSpec((blk, 1), lambda i: (i, 0)),
            pl.BlockSpec((blk, 1), lambda i: (i, 0)),
        ],
        out_specs=[
            pl.BlockSpec((blk, 1), lambda i: (i, 0)),
            pl.BlockSpec((blk, 1), lambda i: (i, 0)),
        ],
        out_shape=[
            jax.ShapeDtypeStruct((rows, 1), jnp.float32),
            jax.ShapeDtypeStruct((rows, 1), jnp.float32),
        ],
    )(W_ih, W_hh, x.reshape(dx, 1), h0.reshape(_H, 1),
      b_ih.reshape(rows, 1), b_hh.reshape(rows, 1))


def _hnew_body(gi_ref, gh_ref, h_ref, o_ref):
    gi = gi_ref[...]
    gh = gh_ref[...]
    h0 = h_ref[...]
    r = jax.nn.sigmoid(gi[0:_H] + gh[0:_H])
    z = jax.nn.sigmoid(gi[_H:2 * _H] + gh[_H:2 * _H])
    n = jnp.tanh(gi[2 * _H:3 * _H] + r * gh[2 * _H:3 * _H])
    o_ref[...] = (1.0 - z) * n + z * h0


def _compute_hnew(gi, gh, h0):
    return pl.pallas_call(
        _hnew_body,
        out_shape=jax.ShapeDtypeStruct((_H, 1), jnp.float32),
    )(gi, gh, h0.reshape(_H, 1))


def _out_body(p_ref, wo_ref, bo_ref, beta_ref, o_ref):
    pv = jax.lax.dot_general(
        p_ref[...], wo_ref[...], (((1,), (1,)), ((), ())),
        preferred_element_type=jnp.float32) + bo_ref[...]
    s = pv * beta_ref[...]
    m = jnp.max(s)
    e = s - m
    o_ref[...] = e - jnp.log(jnp.sum(jnp.exp(e)))


def _compute_out(pred_in, W_o, b_o, beta):
    d_out = W_o.shape[0]
    return pl.pallas_call(
        _out_body,
        out_shape=jax.ShapeDtypeStruct((1, d_out), jnp.float32),
    )(pred_in, W_o, b_o.reshape(1, d_out), beta.reshape(1, d_out))


def kernel(feature, beta, time, fs, hs, ts, father, W_ih, W_hh, b_ih, b_hh,
           W_o, b_o):
    n = ts.shape[0]
    t = jnp.float32(time)
    fs2 = fs.reshape(n, _D_IN)
    hs_flat = hs.reshape(n, _H)
    h0 = hs_flat[-1]

    alpha = _compute_alpha(fs2, feature).reshape(n)

    n_pad = ((n + 1023) // 1024) * 1024
    pad = n_pad - n
    alpha_pad = jnp.concatenate(
        [alpha, jnp.full((pad,), _NEG_INF, jnp.float32)]).reshape(-1, 128)
    ts_pad = jnp.concatenate(
        [ts, jnp.zeros((pad,), jnp.float32)]).reshape(-1, 128)
    t_arr = jnp.full((1, 1), t, jnp.float32)

    w = _compute_weights(alpha_pad, ts_pad, t_arr).reshape(n_pad)[:n]
    attn_h = _compute_attn(w.reshape(n, 1), hs_flat).reshape(_H)

    x = jnp.concatenate([feature, beta, father])
    gi, gh = _compute_gates(W_ih, W_hh, x, h0, b_ih, b_hh)
    h_new = _compute_hnew(gi, gh, h0).reshape(_H)

    length = jnp.full((1,), jnp.float32(_TOPK), jnp.float32)
    pred_in = jnp.concatenate([feature, attn_h, h0, length]).reshape(1, -1)
    out = _compute_out(pred_in, W_o, b_o, beta)

    fs_new = jnp.concatenate([fs, feature])
    hs_new = jnp.concatenate([hs, h_new.reshape(1, 1, _H)], axis=0)
    ts_new = jnp.concatenate([ts, jnp.full((1,), t, jnp.float32)])
    return (out, fs_new, hs_new, ts_new)


# speedup vs baseline: 1.0771x; 1.0771x over previous
import functools
import math

import jax
import jax.numpy as jnp
import numpy as np
from jax import lax
from jax.experimental import pallas as pl
from jax.experimental.pallas import tpu as pltpu
from jax.experimental.pallas import tpu_sc as plsc

_D_IN = 512
_H = 1024
_TOPK = 1024
_DECAY = 0.99
_NEG_INF = np.float32(-np.inf)
_INT_MIN = np.int32(-2147483648)


def _alpha_body(fs_ref, f_ref, o_ref):
    o_ref[...] = jax.lax.dot_general(
        fs_ref[...], f_ref[...], (((1,), (0,)), ((), ())),
        preferred_element_type=jnp.float32)


def _compute_alpha(fs2, feature):
    n = fs2.shape[0]
    blk = 2000
    return pl.pallas_call(
        _alpha_body,
        grid=(n // blk,),
        in_specs=[
            pl.BlockSpec((blk, _D_IN), lambda i: (i, 0)),
            pl.BlockSpec((_D_IN, 1), lambda i: (0, 0)),
        ],
        out_specs=pl.BlockSpec((blk, 1), lambda i: (i, 0)),
        out_shape=jax.ShapeDtypeStruct((n, 1), jnp.float32),
    )(fs2, feature.reshape(_D_IN, 1))


def _weights_body(a_ref, ts_ref, t_ref, w_ref):
    a = a_ref[...]
    rows, lanes = a.shape
    k = _TOPK

    u = jax.lax.bitcast_convert_type(a, jnp.int32)
    key = jnp.where(u < 0, u ^ np.int32(0x7FFFFFFF), u)

    p_u = np.int32(0)
    for b in range(31, -1, -1):
        c_u = p_u | np.int32((1 << b) - (1 << 32 if b == 31 else 0))
        c_s = c_u ^ _INT_MIN
        cnt = jnp.sum((key >= c_s).astype(jnp.int32))
        p_u = jnp.where(cnt >= k, c_u, p_u)
    thr = p_u ^ _INT_MIN

    gt = key > thr
    eq = key == thr
    n_gt = jnp.sum(gt.astype(jnp.int32))
    r = k - n_gt

    idx = (jax.lax.broadcasted_iota(jnp.int32, (rows, lanes), 0) * lanes
           + jax.lax.broadcasted_iota(jnp.int32, (rows, lanes), 1))

    tie_idx = np.int32(0)
    for b in range(14, -1, -1):
        c = tie_idx | np.int32(1 << b)
        cnt = jnp.sum((eq & (idx < c)).astype(jnp.int32))
        tie_idx = jnp.where(cnt < r, c, tie_idx)
    sel = gt | (eq & (idx <= tie_idx))

    t = t_ref[0, 0]
    decay = jnp.exp((t - ts_ref[...]) * np.float32(math.log(_DECAY)))
    s = jnp.where(sel, a * decay, _NEG_INF)
    m = jnp.max(s)
    e = jnp.where(sel, jnp.exp(s - m), np.float32(0.0))
    w_ref[...] = e / jnp.sum(e)


def _compute_weights(alpha_pad, ts_pad, t_arr):
    rows = alpha_pad.shape[0]
    return pl.pallas_call(
        _weights_body,
        in_specs=[
            pl.BlockSpec((rows, 128), lambda: (0, 0)),
            pl.BlockSpec((rows, 128), lambda: (0, 0)),
            pl.BlockSpec(memory_space=pltpu.SMEM),
        ],
        out_specs=pl.BlockSpec((rows, 128), lambda: (0, 0)),
        out_shape=jax.ShapeDtypeStruct((rows, 128), jnp.float32),
    )(alpha_pad, ts_pad, t_arr)


_NC = 2
_NS = 16
_NW = _NC * _NS
_CH = 640
_G = 16


def _sc_attn_body(w_hbm, hs_hbm, out_hbm, w_v, idx_v, wb_v, rows_v, acc_v,
                  part_v, shared_parts, sem):
    c_idx = lax.axis_index("c")
    s_idx = lax.axis_index("s")
    wid = s_idx * _NC + c_idx
    base = wid * _CH
    lane = lax.iota(jnp.int32, 16)
    zf = jnp.zeros((16,), jnp.float32)
    zi = jnp.zeros((16,), jnp.int32)

    pltpu.sync_copy(w_hbm.at[pl.ds(base, _CH)], w_v)
    for v in range(_CH // 16 + 1):
        idx_v[pl.ds(v * 16, 16)] = zi
        wb_v[pl.ds(v * 16, 16)] = zf
    for h in range(_H // 16):
        acc_v[pl.ds(h * 16, 16)] = zf

    base_v = lax.broadcast_in_dim(base, (16,), ())

    def compact(v, cnt):
        wv = w_v[pl.ds(v * 16, 16)]
        selv = wv > 0.0
        gidx = lane + base_v + v * 16
        cnt_v = lax.broadcast_in_dim(cnt, (16,), ())
        pos = cnt_v + plsc.cumsum(selv.astype(jnp.int32)) - 1
        plsc.store_scatter(idx_v, [pos], gidx, mask=selv)
        plsc.store_scatter(wb_v, [pos], wv, mask=selv)
        return cnt + jnp.sum(selv.astype(jnp.int32))

    cnt = lax.fori_loop(0, _CH // 16, compact, wid * 0)

    nch = (cnt + (_G - 1)) // _G

    def chunk(g, carry):
        idxvec = idx_v[pl.ds(g * _G, _G)]
        pltpu.async_copy(hs_hbm.at[idxvec], rows_v, sem).wait()
        wch = wb_v[pl.ds(g * _G, _G)]
        wjs = [lax.broadcast_in_dim(jnp.sum(jnp.where(lane == j, wch, zf)),
                                    (16,), ())
               for j in range(_G)]

        def hstep(h, c2):
            accv = acc_v[pl.ds(h * 16, 16)]
            for j in range(_G):
                accv = accv + wjs[j] * rows_v[j, pl.ds(h * 16, 16)]
            acc_v[pl.ds(h * 16, 16)] = accv
            return c2

        lax.fori_loop(0, _H // 16, hstep, np.int32(0))
        return carry

    lax.fori_loop(0, nch, chunk, np.int32(0))

    pltpu.sync_copy(acc_v, shared_parts.at[s_idx])
    plsc.subcore_barrier()

    rb = s_idx * 64
    accs = [zf, zf, zf, zf]
    for s2 in range(_NS):
        pltpu.sync_copy(shared_parts.at[s2, pl.ds(rb, 64)], part_v)
        for r in range(4):
            accs[r] = accs[r] + part_v[pl.ds(r * 16, 16)]
    for r in range(4):
        part_v[pl.ds(r * 16, 16)] = accs[r]
    pltpu.sync_copy(part_v, out_hbm.at[pl.ds(c_idx * _H + rb, 64)])


def _compute_attn_sc(w_flat, hs_flat):
    f = pl.kernel(
        _sc_attn_body,
        out_type=jax.ShapeDtypeStruct((_NC * _H,), jnp.float32),
        mesh=plsc.VectorSubcoreMesh(core_axis_name="c", subcore_axis_name="s",
                                    num_cores=_NC, num_subcores=_NS),
        compiler_params=pltpu.CompilerParams(needs_layout_passes=False),
        scratch_types=[
            pltpu.VMEM((_CH,), jnp.float32),
            pltpu.VMEM((_CH + 16,), jnp.int32),
            pltpu.VMEM((_CH + 16,), jnp.float32),
            pltpu.VMEM((_G, _H), jnp.float32),
            pltpu.VMEM((_H,), jnp.float32),
            pltpu.VMEM((64,), jnp.float32),
            pltpu.VMEM_SHARED((_NS, _H), jnp.float32),
            pltpu.SemaphoreType.DMA,
        ],
    )
    return f(w_flat, hs_flat)


def _gates_body(wi_ref, wh_ref, x_ref, h_ref, bi_ref, bh_ref, gi_ref, gh_ref):
    gi_ref[...] = jax.lax.dot_general(
        wi_ref[...], x_ref[...], (((1,), (0,)), ((), ())),
        preferred_element_type=jnp.float32) + bi_ref[...]
    gh_ref[...] = jax.lax.dot_general(
        wh_ref[...], h_ref[...], (((1,), (0,)), ((), ())),
        preferred_element_type=jnp.float32) + bh_ref[...]


def _compute_gates(W_ih, W_hh, x, h0, b_ih, b_hh):
    rows = W_ih.shape[0]
    dx = W_ih.shape[1]
    blk = 512
    return pl.pallas_call(
        _gates_body,
        grid=(rows // blk,),
        in_specs=[
            pl.BlockSpec((blk, dx), lambda i: (i, 0)),
            pl.BlockSpec((blk, _H), lambda i: (i, 0)),
            pl.BlockSpec((dx, 1), lambda i: (0, 0)),
            pl.BlockSpec((_H, 1), lambda i: (0, 0)),
            pl.BlockSpec((blk, 1), lambda i: (i, 0)),
            pl.BlockSpec((blk, 1), lambda i: (i, 0)),
        ],
        out_specs=[
            pl.BlockSpec((blk, 1), lambda i: (i, 0)),
            pl.BlockSpec((blk, 1), lambda i: (i, 0)),
        ],
        out_shape=[
            jax.ShapeDtypeStruct((rows, 1), jnp.float32),
            jax.ShapeDtypeStruct((rows, 1), jnp.float32),
        ],
    )(W_ih, W_hh, x.reshape(dx, 1), h0.reshape(_H, 1),
      b_ih.reshape(rows, 1), b_hh.reshape(rows, 1))


def _hnew_body(gi_ref, gh_ref, h_ref, o_ref):
    gi = gi_ref[...]
    gh = gh_ref[...]
    h0 = h_ref[...]
    r = jax.nn.sigmoid(gi[0:_H] + gh[0:_H])
    z = jax.nn.sigmoid(gi[_H:2 * _H] + gh[_H:2 * _H])
    n = jnp.tanh(gi[2 * _H:3 * _H] + r * gh[2 * _H:3 * _H])
    o_ref[...] = (1.0 - z) * n + z * h0


def _compute_hnew(gi, gh, h0):
    return pl.pallas_call(
        _hnew_body,
        out_shape=jax.ShapeDtypeStruct((_H, 1), jnp.float32),
    )(gi, gh, h0.reshape(_H, 1))


def _out_body(p_ref, wo_ref, bo_ref, beta_ref, o_ref):
    pv = jax.lax.dot_general(
        p_ref[...], wo_ref[...], (((1,), (1,)), ((), ())),
        preferred_element_type=jnp.float32) + bo_ref[...]
    s = pv * beta_ref[...]
    m = jnp.max(s)
    e = s - m
    o_ref[...] = e - jnp.log(jnp.sum(jnp.exp(e)))


def _compute_out(pred_in, W_o, b_o, beta):
    d_out = W_o.shape[0]
    return pl.pallas_call(
        _out_body,
        out_shape=jax.ShapeDtypeStruct((1, d_out), jnp.float32),
    )(pred_in, W_o, b_o.reshape(1, d_out), beta.reshape(1, d_out))


def kernel(feature, beta, time, fs, hs, ts, father, W_ih, W_hh, b_ih, b_hh,
           W_o, b_o):
    n = ts.shape[0]
    t = jnp.float32(time)
    fs2 = fs.reshape(n, _D_IN)
    hs_flat = hs.reshape(n, _H)
    h0 = hs_flat[-1]

    alpha = _compute_alpha(fs2, feature).reshape(n)

    n_pad = ((n + 1023) // 1024) * 1024
    pad = n_pad - n
    alpha_pad = jnp.concatenate(
        [alpha, jnp.full((pad,), _NEG_INF, jnp.float32)]).reshape(-1, 128)
    ts_pad = jnp.concatenate(
        [ts, jnp.zeros((pad,), jnp.float32)]).reshape(-1, 128)
    t_arr = jnp.full((1, 1), t, jnp.float32)

    w = _compute_weights(alpha_pad, ts_pad, t_arr).reshape(n_pad)
    attn_h = _compute_attn_sc(w, hs_flat).reshape(_NC, _H).sum(axis=0)

    x = jnp.concatenate([feature, beta, father])
    gi, gh = _compute_gates(W_ih, W_hh, x, h0, b_ih, b_hh)
    h_new = _compute_hnew(gi, gh, h0).reshape(_H)

    length = jnp.full((1,), jnp.float32(_TOPK), jnp.float32)
    pred_in = jnp.concatenate([feature, attn_h, h0, length]).reshape(1, -1)
    out = _compute_out(pred_in, W_o, b_o, beta)

    fs_new = jnp.concatenate([fs, feature])
    hs_new = jnp.concatenate([hs, h_new.reshape(1, 1, _H)], axis=0)
    ts_new = jnp.concatenate([ts, jnp.full((1,), t, jnp.float32)])
    return (out, fs_new, hs_new, ts_new)


# speedup vs baseline: 1.2360x; 1.1476x over previous
import functools
import math

import jax
import jax.numpy as jnp
import numpy as np
from jax import lax
from jax.experimental import pallas as pl
from jax.experimental.pallas import tpu as pltpu
from jax.experimental.pallas import tpu_sc as plsc

_D_IN = 512
_H = 1024
_TOPK = 1024
_DECAY = 0.99
_NEG_INF = np.float32(-np.inf)
_INT_MIN = np.int32(-2147483648)


def _alpha_body(fs_ref, f_ref, o_ref):
    o_ref[...] = jax.lax.dot_general(
        fs_ref[...], f_ref[...], (((1,), (0,)), ((), ())),
        preferred_element_type=jnp.float32)


def _compute_alpha(fs2, feature):
    n = fs2.shape[0]
    blk = 2000
    return pl.pallas_call(
        _alpha_body,
        grid=(n // blk,),
        in_specs=[
            pl.BlockSpec((blk, _D_IN), lambda i: (i, 0)),
            pl.BlockSpec((_D_IN, 1), lambda i: (0, 0)),
        ],
        out_specs=pl.BlockSpec((blk, 1), lambda i: (i, 0)),
        out_shape=jax.ShapeDtypeStruct((n, 1), jnp.float32),
    )(fs2, feature.reshape(_D_IN, 1))


def _weights_body(a_ref, ts_ref, t_ref, w_ref):
    a = a_ref[...]
    rows, lanes = a.shape
    k = _TOPK

    u = jax.lax.bitcast_convert_type(a, jnp.int32)
    key = jnp.where(u < 0, u ^ np.int32(0x7FFFFFFF), u)

    p_u = np.int32(0)
    for b in range(31, -1, -1):
        c_u = p_u | np.int32((1 << b) - (1 << 32 if b == 31 else 0))
        c_s = c_u ^ _INT_MIN
        cnt = jnp.sum((key >= c_s).astype(jnp.int32))
        p_u = jnp.where(cnt >= k, c_u, p_u)
    thr = p_u ^ _INT_MIN

    gt = key > thr
    eq = key == thr
    n_gt = jnp.sum(gt.astype(jnp.int32))
    r = k - n_gt

    idx = (jax.lax.broadcasted_iota(jnp.int32, (rows, lanes), 0) * lanes
           + jax.lax.broadcasted_iota(jnp.int32, (rows, lanes), 1))

    tie_idx = np.int32(0)
    for b in range(14, -1, -1):
        c = tie_idx | np.int32(1 << b)
        cnt = jnp.sum((eq & (idx < c)).astype(jnp.int32))
        tie_idx = jnp.where(cnt < r, c, tie_idx)
    sel = gt | (eq & (idx <= tie_idx))

    t = t_ref[0, 0]
    decay = jnp.exp((t - ts_ref[...]) * np.float32(math.log(_DECAY)))
    s = jnp.where(sel, a * decay, _NEG_INF)
    m = jnp.max(s)
    e = jnp.where(sel, jnp.exp(s - m), np.float32(0.0))
    w_ref[...] = e / jnp.sum(e)


def _compute_weights(alpha_pad, ts_pad, t_arr):
    rows = alpha_pad.shape[0]
    return pl.pallas_call(
        _weights_body,
        in_specs=[
            pl.BlockSpec((rows, 128), lambda: (0, 0)),
            pl.BlockSpec((rows, 128), lambda: (0, 0)),
            pl.BlockSpec(memory_space=pltpu.SMEM),
        ],
        out_specs=pl.BlockSpec((rows, 128), lambda: (0, 0)),
        out_shape=jax.ShapeDtypeStruct((rows, 128), jnp.float32),
    )(alpha_pad, ts_pad, t_arr)


_NC = 2
_NS = 16
_NW = _NC * _NS
_CH = 640
_G = 16


def _sc_attn_body(w_hbm, hs_hbm, out_hbm, w_v, idx_v, wb_v, rows_v, acc_v,
                  part_v, shared_parts, sem):
    c_idx = lax.axis_index("c")
    s_idx = lax.axis_index("s")
    wid = s_idx * _NC + c_idx
    base = wid * _CH
    lane = lax.iota(jnp.int32, 16)
    zf = jnp.zeros((16,), jnp.float32)
    zi = jnp.zeros((16,), jnp.int32)

    pltpu.sync_copy(w_hbm.at[pl.ds(base, _CH)], w_v)
    for v in range(_CH // 16 + 1):
        idx_v[pl.ds(v * 16, 16)] = zi
        wb_v[pl.ds(v * 16, 16)] = zf
    for h in range(_H // 16):
        acc_v[pl.ds(h * 16, 16)] = zf

    base_v = lax.broadcast_in_dim(base, (16,), ())

    def compact(v, cnt):
        wv = w_v[pl.ds(v * 16, 16)]
        selv = wv > 0.0
        gidx = lane + base_v + v * 16
        cnt_v = lax.broadcast_in_dim(cnt, (16,), ())
        pos = cnt_v + plsc.cumsum(selv.astype(jnp.int32)) - 1
        plsc.store_scatter(idx_v, [pos], gidx, mask=selv)
        plsc.store_scatter(wb_v, [pos], wv, mask=selv)
        return cnt + jnp.sum(selv.astype(jnp.int32))

    cnt = lax.fori_loop(0, _CH // 16, compact, wid * 0)

    nch = (cnt + (_G - 1)) // _G

    def chunk(g, carry):
        idxvec = idx_v[pl.ds(g * _G, _G)]
        pltpu.async_copy(hs_hbm.at[idxvec], rows_v, sem).wait()
        wch = wb_v[pl.ds(g * _G, _G)]
        wjs = [lax.broadcast_in_dim(jnp.sum(jnp.where(lane == j, wch, zf)),
                                    (16,), ())
               for j in range(_G)]

        def hstep(h, c2):
            accv = acc_v[pl.ds(h * 16, 16)]
            for j in range(_G):
                accv = accv + wjs[j] * rows_v[j, pl.ds(h * 16, 16)]
            acc_v[pl.ds(h * 16, 16)] = accv
            return c2

        lax.fori_loop(0, _H // 16, hstep, np.int32(0))
        return carry

    lax.fori_loop(0, nch, chunk, np.int32(0))

    pltpu.sync_copy(acc_v, shared_parts.at[s_idx])
    plsc.subcore_barrier()

    rb = s_idx * 64
    accs = [zf, zf, zf, zf]
    for s2 in range(_NS):
        pltpu.sync_copy(shared_parts.at[s2, pl.ds(rb, 64)], part_v)
        for r in range(4):
            accs[r] = accs[r] + part_v[pl.ds(r * 16, 16)]
    for r in range(4):
        part_v[pl.ds(r * 16, 16)] = accs[r]
    pltpu.sync_copy(part_v, out_hbm.at[pl.ds(c_idx * _H + rb, 64)])


def _compute_attn_sc(w_flat, hs_flat):
    f = pl.kernel(
        _sc_attn_body,
        out_type=jax.ShapeDtypeStruct((_NC * _H,), jnp.float32),
        mesh=plsc.VectorSubcoreMesh(core_axis_name="c", subcore_axis_name="s",
                                    num_cores=_NC, num_subcores=_NS),
        compiler_params=pltpu.CompilerParams(needs_layout_passes=False),
        scratch_types=[
            pltpu.VMEM((_CH,), jnp.float32),
            pltpu.VMEM((_CH + 16,), jnp.int32),
            pltpu.VMEM((_CH + 16,), jnp.float32),
            pltpu.VMEM((_G, _H), jnp.float32),
            pltpu.VMEM((_H,), jnp.float32),
            pltpu.VMEM((64,), jnp.float32),
            pltpu.VMEM_SHARED((_NS, _H), jnp.float32),
            pltpu.SemaphoreType.DMA,
        ],
    )
    return f(w_flat, hs_flat)


def _gates_body(wi_ref, wh_ref, x_ref, h_ref, bi_ref, bh_ref, gi_ref, gh_ref):
    gi_ref[...] = jax.lax.dot_general(
        wi_ref[...], x_ref[...], (((1,), (0,)), ((), ())),
        preferred_element_type=jnp.float32) + bi_ref[...]
    gh_ref[...] = jax.lax.dot_general(
        wh_ref[...], h_ref[...], (((1,), (0,)), ((), ())),
        preferred_element_type=jnp.float32) + bh_ref[...]


def _compute_gates(W_ih, W_hh, x, h0, b_ih, b_hh):
    rows = W_ih.shape[0]
    dx = W_ih.shape[1]
    blk = 512
    return pl.pallas_call(
        _gates_body,
        grid=(rows // blk,),
        in_specs=[
            pl.BlockSpec((blk, dx), lambda i: (i, 0)),
            pl.BlockSpec((blk, _H), lambda i: (i, 0)),
            pl.BlockSpec((dx, 1), lambda i: (0, 0)),
            pl.BlockSpec((_H, 1), lambda i: (0, 0)),
            pl.BlockSpec((blk, 1), lambda i: (i, 0)),
            pl.BlockSpec((blk, 1), lambda i: (i, 0)),
        ],
        out_specs=[
            pl.BlockSpec((blk, 1), lambda i: (i, 0)),
            pl.BlockSpec((blk, 1), lambda i: (i, 0)),
        ],
        out_shape=[
            jax.ShapeDtypeStruct((rows, 1), jnp.float32),
            jax.ShapeDtypeStruct((rows, 1), jnp.float32),
        ],
    )(W_ih, W_hh, x.reshape(dx, 1), h0.reshape(_H, 1),
      b_ih.reshape(rows, 1), b_hh.reshape(rows, 1))


def _hnew_body(gi_ref, gh_ref, h_ref, o_ref):
    gi = gi_ref[...]
    gh = gh_ref[...]
    h0 = h_ref[...]
    r = jax.nn.sigmoid(gi[0:_H] + gh[0:_H])
    z = jax.nn.sigmoid(gi[_H:2 * _H] + gh[_H:2 * _H])
    n = jnp.tanh(gi[2 * _H:3 * _H] + r * gh[2 * _H:3 * _H])
    o_ref[...] = (1.0 - z) * n + z * h0


def _compute_hnew(gi, gh, h0):
    return pl.pallas_call(
        _hnew_body,
        out_shape=jax.ShapeDtypeStruct((_H, 1), jnp.float32),
    )(gi, gh, h0.reshape(_H, 1))


def _out_body(p_ref, wo_ref, bo_ref, beta_ref, o_ref):
    pv = jax.lax.dot_general(
        p_ref[...], wo_ref[...], (((1,), (1,)), ((), ())),
        preferred_element_type=jnp.float32) + bo_ref[...]
    s = pv * beta_ref[...]
    m = jnp.max(s)
    e = s - m
    o_ref[...] = e - jnp.log(jnp.sum(jnp.exp(e)))


def _compute_out(pred_in, W_o, b_o, beta):
    d_out = W_o.shape[0]
    return pl.pallas_call(
        _out_body,
        out_shape=jax.ShapeDtypeStruct((1, d_out), jnp.float32),
    )(pred_in, W_o, b_o.reshape(1, d_out), beta.reshape(1, d_out))


def _concat_body(hs_ref, fs_ref, hn_ref, ft_ref, oh_ref, of_ref):
    i = pl.program_id(0)

    @pl.when(i < 10)
    def _():
        oh_ref[...] = hs_ref[...]
        of_ref[...] = fs_ref[...]

    @pl.when(i == 10)
    def _():
        oh_ref[...] = jnp.broadcast_to(hn_ref[...], oh_ref.shape)
        of_ref[...] = jnp.broadcast_to(ft_ref[...], of_ref.shape)


def _concat_rows(hs2, fs2, h_new, feature):
    n, h = hs2.shape
    d = fs2.shape[1]
    blk = 2000
    return pl.pallas_call(
        _concat_body,
        grid=(n // blk + 1,),
        in_specs=[
            pl.BlockSpec((blk, h), lambda i: (jnp.minimum(i, 9), 0)),
            pl.BlockSpec((blk, d), lambda i: (jnp.minimum(i, 9), 0)),
            pl.BlockSpec((1, h), lambda i: (0, 0)),
            pl.BlockSpec((1, d), lambda i: (0, 0)),
        ],
        out_specs=[
            pl.BlockSpec((blk, h), lambda i: (i, 0)),
            pl.BlockSpec((blk, d), lambda i: (i, 0)),
        ],
        out_shape=[
            jax.ShapeDtypeStruct((n + 1, h), jnp.float32),
            jax.ShapeDtypeStruct((n + 1, d), jnp.float32),
        ],
    )(hs2, fs2, h_new.reshape(1, h), feature.reshape(1, d))


def kernel(feature, beta, time, fs, hs, ts, father, W_ih, W_hh, b_ih, b_hh,
           W_o, b_o):
    n = ts.shape[0]
    t = jnp.float32(time)
    fs2 = fs.reshape(n, _D_IN)
    hs_flat = hs.reshape(n, _H)
    h0 = hs_flat[-1]

    alpha = _compute_alpha(fs2, feature).reshape(n)

    n_pad = ((n + 1023) // 1024) * 1024
    pad = n_pad - n
    alpha_pad = jnp.concatenate(
        [alpha, jnp.full((pad,), _NEG_INF, jnp.float32)]).reshape(-1, 128)
    ts_pad = jnp.concatenate(
        [ts, jnp.zeros((pad,), jnp.float32)]).reshape(-1, 128)
    t_arr = jnp.full((1, 1), t, jnp.float32)

    w = _compute_weights(alpha_pad, ts_pad, t_arr).reshape(n_pad)
    attn_h = _compute_attn_sc(w, hs_flat).reshape(_NC, _H).sum(axis=0)

    x = jnp.concatenate([feature, beta, father])
    gi, gh = _compute_gates(W_ih, W_hh, x, h0, b_ih, b_hh)
    h_new = _compute_hnew(gi, gh, h0).reshape(_H)

    length = jnp.full((1,), jnp.float32(_TOPK), jnp.float32)
    pred_in = jnp.concatenate([feature, attn_h, h0, length]).reshape(1, -1)
    out = _compute_out(pred_in, W_o, b_o, beta)

    hs_new2, fs_new2 = _concat_rows(hs_flat, fs2, h_new, feature)
    fs_new = fs_new2.reshape((n + 1) * _D_IN)
    hs_new = hs_new2.reshape(n + 1, 1, _H)
    ts_new = jnp.concatenate([ts, jnp.full((1,), t, jnp.float32)])
    return (out, fs_new, hs_new, ts_new)


# speedup vs baseline: 1.8630x; 1.5072x over previous
import functools
import math

import jax
import jax.numpy as jnp
import numpy as np
from jax import lax
from jax.experimental import pallas as pl
from jax.experimental.pallas import tpu as pltpu
from jax.experimental.pallas import tpu_sc as plsc

_D_IN = 512
_H = 1024
_TOPK = 1024
_DECAY = 0.99
_NEG_INF = np.float32(-np.inf)
_INT_MIN = np.int32(-2147483648)


def _alpha_body(fs_ref, f_ref, o_ref):
    o_ref[...] = jax.lax.dot_general(
        fs_ref[...], f_ref[...], (((1,), (0,)), ((), ())),
        preferred_element_type=jnp.float32)


def _compute_alpha(fs2, feature):
    n = fs2.shape[0]
    blk = 2000
    return pl.pallas_call(
        _alpha_body,
        grid=(n // blk,),
        in_specs=[
            pl.BlockSpec((blk, _D_IN), lambda i: (i, 0)),
            pl.BlockSpec((_D_IN, 1), lambda i: (0, 0)),
        ],
        out_specs=pl.BlockSpec((blk, 1), lambda i: (i, 0)),
        out_shape=jax.ShapeDtypeStruct((n, 1), jnp.float32),
    )(fs2, feature.reshape(_D_IN, 1))


def _weights_body(a_ref, ts_ref, t_ref, w_ref):
    a = a_ref[...]
    rows, lanes = a.shape
    k = _TOPK

    u = jax.lax.bitcast_convert_type(a, jnp.int32)
    key = jnp.where(u < 0, u ^ np.int32(0x7FFFFFFF), u)

    p_u = np.int32(0)
    for b in range(31, -1, -1):
        c_u = p_u | np.int32((1 << b) - (1 << 32 if b == 31 else 0))
        c_s = c_u ^ _INT_MIN
        cnt = jnp.sum((key >= c_s).astype(jnp.int32))
        p_u = jnp.where(cnt >= k, c_u, p_u)
    thr = p_u ^ _INT_MIN

    gt = key > thr
    eq = key == thr
    n_gt = jnp.sum(gt.astype(jnp.int32))
    r = k - n_gt

    idx = (jax.lax.broadcasted_iota(jnp.int32, (rows, lanes), 0) * lanes
           + jax.lax.broadcasted_iota(jnp.int32, (rows, lanes), 1))

    tie_idx = np.int32(0)
    for b in range(14, -1, -1):
        c = tie_idx | np.int32(1 << b)
        cnt = jnp.sum((eq & (idx < c)).astype(jnp.int32))
        tie_idx = jnp.where(cnt < r, c, tie_idx)
    sel = gt | (eq & (idx <= tie_idx))

    t = t_ref[0, 0]
    decay = jnp.exp((t - ts_ref[...]) * np.float32(math.log(_DECAY)))
    s = jnp.where(sel, a * decay, _NEG_INF)
    m = jnp.max(s)
    e = jnp.where(sel, jnp.exp(s - m), np.float32(0.0))
    w_ref[...] = e / jnp.sum(e)


def _compute_weights(alpha_pad, ts_pad, t_arr):
    rows = alpha_pad.shape[0]
    return pl.pallas_call(
        _weights_body,
        in_specs=[
            pl.BlockSpec((rows, 128), lambda: (0, 0)),
            pl.BlockSpec((rows, 128), lambda: (0, 0)),
            pl.BlockSpec(memory_space=pltpu.SMEM),
        ],
        out_specs=pl.BlockSpec((rows, 128), lambda: (0, 0)),
        out_shape=jax.ShapeDtypeStruct((rows, 128), jnp.float32),
    )(alpha_pad, ts_pad, t_arr)


_NC = 2
_NS = 16
_NW = _NC * _NS
_CH = 640
_G = 16


def _sc_attn_body(w_hbm, hs_hbm, out_hbm, w_v, idx_v, wb_v, rows_v, acc_v,
                  part_v, shared_parts, sem):
    c_idx = lax.axis_index("c")
    s_idx = lax.axis_index("s")
    wid = s_idx * _NC + c_idx
    base = wid * _CH
    lane = lax.iota(jnp.int32, 16)
    zf = jnp.zeros((16,), jnp.float32)
    zi = jnp.zeros((16,), jnp.int32)

    pltpu.sync_copy(w_hbm.at[pl.ds(base, _CH)], w_v)
    for v in range(_CH // 16 + 1):
        idx_v[pl.ds(v * 16, 16)] = zi
        wb_v[pl.ds(v * 16, 16)] = zf
    for h in range(_H // 16):
        acc_v[pl.ds(h * 16, 16)] = zf

    base_v = lax.broadcast_in_dim(base, (16,), ())

    def compact(v, cnt):
        wv = w_v[pl.ds(v * 16, 16)]
        selv = wv > 0.0
        gidx = lane + base_v + v * 16
        cnt_v = lax.broadcast_in_dim(cnt, (16,), ())
        pos = cnt_v + plsc.cumsum(selv.astype(jnp.int32)) - 1
        plsc.store_scatter(idx_v, [pos], gidx, mask=selv)
        plsc.store_scatter(wb_v, [pos], wv, mask=selv)
        return cnt + jnp.sum(selv.astype(jnp.int32))

    cnt = lax.fori_loop(0, _CH // 16, compact, wid * 0)

    nch = (cnt + (_G - 1)) // _G

    def chunk(g, carry):
        idxvec = idx_v[pl.ds(g * _G, _G)]
        pltpu.async_copy(hs_hbm.at[idxvec], rows_v, sem).wait()

        wch = wb_v[pl.ds(g * _G, _G)]
        wjs = [lax.broadcast_in_dim(jnp.sum(jnp.where(lane == j, wch, zf)),
                                    (16,), ())
               for j in range(_G)]

        def hstep(h, c2):
            accv = acc_v[pl.ds(h * 16, 16)]
            for j in range(_G):
                accv = accv + wjs[j] * rows_v[j, 0, pl.ds(h * 16, 16)]
            acc_v[pl.ds(h * 16, 16)] = accv
            return c2

        lax.fori_loop(0, _H // 16, hstep, np.int32(0))
        return carry

    lax.fori_loop(0, nch, chunk, np.int32(0))

    pltpu.sync_copy(acc_v, shared_parts.at[s_idx])
    plsc.subcore_barrier()

    rb = s_idx * 64
    accs = [zf, zf, zf, zf]
    for s2 in range(_NS):
        pltpu.sync_copy(shared_parts.at[s2, pl.ds(rb, 64)], part_v)
        for r in range(4):
            accs[r] = accs[r] + part_v[pl.ds(r * 16, 16)]
    for r in range(4):
        part_v[pl.ds(r * 16, 16)] = accs[r]
    pltpu.sync_copy(part_v, out_hbm.at[pl.ds(c_idx * _H + rb, 64)])


def _compute_attn_sc(w_flat, hs_flat):
    f = pl.kernel(
        _sc_attn_body,
        out_type=jax.ShapeDtypeStruct((_NC * _H,), jnp.float32),
        mesh=plsc.VectorSubcoreMesh(core_axis_name="c", subcore_axis_name="s",
                                    num_cores=_NC, num_subcores=_NS),
        compiler_params=pltpu.CompilerParams(needs_layout_passes=False),
        scratch_types=[
            pltpu.VMEM((_CH,), jnp.float32),
            pltpu.VMEM((_CH + 16,), jnp.int32),
            pltpu.VMEM((_CH + 16,), jnp.float32),
            pltpu.VMEM((_G, 1, _H), jnp.float32),
            pltpu.VMEM((_H,), jnp.float32),
            pltpu.VMEM((64,), jnp.float32),
            pltpu.VMEM_SHARED((_NS, _H), jnp.float32),
            pltpu.SemaphoreType.DMA,
        ],
    )
    return f(w_flat, hs_flat)


def _gates_body(wi_ref, wh_ref, x_ref, h_ref, bi_ref, bh_ref, gi_ref, gh_ref):
    gi_ref[...] = jax.lax.dot_general(
        wi_ref[...], x_ref[...], (((1,), (0,)), ((), ())),
        preferred_element_type=jnp.float32) + bi_ref[...]
    gh_ref[...] = jax.lax.dot_general(
        wh_ref[...], h_ref[...], (((1,), (0,)), ((), ())),
        preferred_element_type=jnp.float32) + bh_ref[...]


def _compute_gates(W_ih, W_hh, x, h0, b_ih, b_hh):
    rows = W_ih.shape[0]
    dx = W_ih.shape[1]
    blk = 512
    return pl.pallas_call(
        _gates_body,
        grid=(rows // blk,),
        in_specs=[
            pl.BlockSpec((blk, dx), lambda i: (i, 0)),
            pl.BlockSpec((blk, _H), lambda i: (i, 0)),
            pl.BlockSpec((dx, 1), lambda i: (0, 0)),
            pl.BlockSpec((_H, 1), lambda i: (0, 0)),
            pl.BlockSpec((blk, 1), lambda i: (i, 0)),
            pl.BlockSpec((blk, 1), lambda i: (i, 0)),
        ],
        out_specs=[
            pl.BlockSpec((blk, 1), lambda i: (i, 0)),
            pl.BlockSpec((blk, 1), lambda i: (i, 0)),
        ],
        out_shape=[
            jax.ShapeDtypeStruct((rows, 1), jnp.float32),
            jax.ShapeDtypeStruct((rows, 1), jnp.float32),
        ],
    )(W_ih, W_hh, x.reshape(dx, 1), h0.reshape(_H, 1),
      b_ih.reshape(rows, 1), b_hh.reshape(rows, 1))


def _hnew_body(gi_ref, gh_ref, h_ref, o_ref):
    gi = gi_ref[...]
    gh = gh_ref[...]
    h0 = h_ref[...]
    r = jax.nn.sigmoid(gi[0:_H] + gh[0:_H])
    z = jax.nn.sigmoid(gi[_H:2 * _H] + gh[_H:2 * _H])
    n = jnp.tanh(gi[2 * _H:3 * _H] + r * gh[2 * _H:3 * _H])
    o_ref[...] = (1.0 - z) * n + z * h0


def _compute_hnew(gi, gh, h0):
    return pl.pallas_call(
        _hnew_body,
        out_shape=jax.ShapeDtypeStruct((_H, 1), jnp.float32),
    )(gi, gh, h0.reshape(_H, 1))


def _out_body(p_ref, wo_ref, bo_ref, beta_ref, o_ref):
    pv = jax.lax.dot_general(
        p_ref[...], wo_ref[...], (((1,), (1,)), ((), ())),
        preferred_element_type=jnp.float32) + bo_ref[...]
    s = pv * beta_ref[...]
    m = jnp.max(s)
    e = s - m
    o_ref[...] = e - jnp.log(jnp.sum(jnp.exp(e)))


def _compute_out(pred_in, W_o, b_o, beta):
    d_out = W_o.shape[0]
    return pl.pallas_call(
        _out_body,
        out_shape=jax.ShapeDtypeStruct((1, d_out), jnp.float32),
    )(pred_in, W_o, b_o.reshape(1, d_out), beta.reshape(1, d_out))


def _concat_body(hs_ref, fs_ref, hn_ref, ft_ref, oh_ref, of_ref):
    i = pl.program_id(0)

    @pl.when(i < 10)
    def _():
        oh_ref[...] = hs_ref[...]
        of_ref[...] = fs_ref[...]

    @pl.when(i == 10)
    def _():
        oh_ref[...] = jnp.broadcast_to(hn_ref[...], oh_ref.shape)
        of_ref[pl.ds(0, _D_IN)] = ft_ref[...]


def _concat_rows(hs3, fs, h_new, feature):
    n = hs3.shape[0]
    blk = 2000
    fblk = blk * _D_IN
    return pl.pallas_call(
        _concat_body,
        grid=(n // blk + 1,),
        in_specs=[
            pl.BlockSpec((blk, 1, _H), lambda i: (jnp.minimum(i, 9), 0, 0)),
            pl.BlockSpec((fblk,), lambda i: (jnp.minimum(i, 9),)),
            pl.BlockSpec((1, 1, _H), lambda i: (0, 0, 0)),
            pl.BlockSpec((_D_IN,), lambda i: (0,)),
        ],
        out_specs=[
            pl.BlockSpec((blk, 1, _H), lambda i: (i, 0, 0)),
            pl.BlockSpec((fblk,), lambda i: (i,)),
        ],
        out_shape=[
            jax.ShapeDtypeStruct((n + 1, 1, _H), jnp.float32),
            jax.ShapeDtypeStruct(((n + 1) * _D_IN,), jnp.float32),
        ],
    )(hs3, fs, h_new.reshape(1, 1, _H), feature)


def kernel(feature, beta, time, fs, hs, ts, father, W_ih, W_hh, b_ih, b_hh,
           W_o, b_o):
    n = ts.shape[0]
    t = jnp.float32(time)
    fs2 = fs.reshape(n, _D_IN)
    h0 = hs[-1].reshape(_H)

    alpha = _compute_alpha(fs2, feature).reshape(n)

    n_pad = ((n + 1023) // 1024) * 1024
    pad = n_pad - n
    alpha_pad = jnp.concatenate(
        [alpha, jnp.full((pad,), _NEG_INF, jnp.float32)]).reshape(-1, 128)
    ts_pad = jnp.concatenate(
        [ts, jnp.zeros((pad,), jnp.float32)]).reshape(-1, 128)
    t_arr = jnp.full((1, 1), t, jnp.float32)

    w = _compute_weights(alpha_pad, ts_pad, t_arr).reshape(n_pad)
    attn_h = _compute_attn_sc(w, hs).reshape(_NC, _H).sum(axis=0)

    x = jnp.concatenate([feature, beta, father])
    gi, gh = _compute_gates(W_ih, W_hh, x, h0, b_ih, b_hh)
    h_new = _compute_hnew(gi, gh, h0).reshape(_H)

    length = jnp.full((1,), jnp.float32(_TOPK), jnp.float32)
    pred_in = jnp.concatenate([feature, attn_h, h0, length]).reshape(1, -1)
    out = _compute_out(pred_in, W_o, b_o, beta)

    hs_new, fs_new = _concat_rows(hs, fs, h_new, feature)
    ts_new = jnp.concatenate([ts, jnp.full((1,), t, jnp.float32)])
    return (out, fs_new, hs_new, ts_new)
